# baseline (speedup 1.0000x reference)
import functools

import jax
import jax.numpy as jnp
from jax import lax
from jax.experimental import pallas as pl
from jax.experimental.pallas import tpu as pltpu
from jax.experimental.pallas import tpu_sc as plsc

MARGIN = 0.6
NSC = 4
LANES = 16
SC_CORES = 2
SC_SUBCORES = 16


def _tc_body(val_ref, idxo_ref, syb_ref, att_ref, out_ref):
    blocks = att_ref.shape[0]
    v = syb_ref.shape[1]

    vrow = val_ref[0]
    iorow = idxo_ref[0]

    citer = lax.broadcasted_iota(jnp.int32, (v, v), 0)
    ovt = (jnp.broadcast_to(vrow, (v, v)) == citer).astype(jnp.float32)
    oit = (jnp.broadcast_to(iorow, (v, v)) == citer).astype(jnp.float32)

    eq = lax.dot_general(ovt, ovt, (((0,), (0,)), ((), ())),
                         preferred_element_type=jnp.float32)
    lprime = lax.broadcasted_iota(jnp.int32, (v, v), 1)
    first = jnp.min(jnp.where(eq > 0.5, lprime, v), axis=1, keepdims=True)
    lidx = lax.broadcasted_iota(jnp.int32, (v, 1), 0)
    m = (first == lidx).astype(jnp.float32)

    sybsign = 1.0 - 2.0 * syb_ref[0].astype(jnp.float32)
    gs = lax.dot_general(oit, sybsign, (((0,), (0,)), ((), ())),
                         preferred_element_type=jnp.float32)

    partial = jnp.float32(0.0)
    for b in range(blocks):
        d = jnp.sum(att_ref[b, 0] * gs, axis=1, keepdims=True)
        partial += jnp.sum(jnp.maximum(d + MARGIN, 0.0) * m)
    nfirst = jnp.sum(m)
    partial += blocks * (v - nfirst) * MARGIN

    tile = jnp.full((8, 128), partial, dtype=jnp.float32)
    i = pl.program_id(0)

    @pl.when(i == 0)
    def _init():
        out_ref[...] = tile

    @pl.when(i > 0)
    def _acc():
        out_ref[...] += tile


def _tc_partial(idx_of_objs, valid2all, syb_graph, att_weights, nsc):
    blocks, bsz, v, s = att_weights.shape
    nb = bsz - nsc
    val3 = valid2all.reshape(bsz, 1, v)
    idx3 = idx_of_objs.reshape(bsz, 1, v)

    out = pl.pallas_call(
        _tc_body,
        grid=(nb,),
        in_specs=[
            pl.BlockSpec((1, 1, v), lambda i: (i + nsc, 0, 0)),
            pl.BlockSpec((1, 1, v), lambda i: (i + nsc, 0, 0)),
            pl.BlockSpec((1, v, s), lambda i: (i + nsc, 0, 0)),
            pl.BlockSpec((blocks, 1, v, s), lambda i: (0, i + nsc, 0, 0)),
        ],
        out_specs=pl.BlockSpec((8, 128), lambda i: (0, 0)),
        out_shape=jax.ShapeDtypeStruct((8, 128), jnp.float32),
    )(val3, idx3, syb_graph, att_weights)
    return out[0, 0]


def _prep_body(val_ref, idxo_ref, syb_ref, m_ref, gs_ref):
    v = syb_ref.shape[1]
    vrow = val_ref[0]
    iorow = idxo_ref[0]

    citer = lax.broadcasted_iota(jnp.int32, (v, v), 0)
    ovt = (jnp.broadcast_to(vrow, (v, v)) == citer).astype(jnp.float32)
    oit = (jnp.broadcast_to(iorow, (v, v)) == citer).astype(jnp.float32)

    eq = lax.dot_general(ovt, ovt, (((0,), (0,)), ((), ())),
                         preferred_element_type=jnp.float32)
    lp0 = lax.broadcasted_iota(jnp.int32, (v, v), 0)
    first = jnp.min(jnp.where(eq > 0.5, lp0, v), axis=0, keepdims=True)
    laneio = lax.broadcasted_iota(jnp.int32, (1, v), 1)
    m_ref[0] = (first == laneio).astype(jnp.float32)

    sybsign = 1.0 - 2.0 * syb_ref[0].astype(jnp.float32)
    gs_ref[0] = lax.dot_general(oit, sybsign, (((0,), (0,)), ((), ())),
                                preferred_element_type=jnp.float32)


def _tc_prep(idx_of_objs, valid2all, syb_graph, att_weights, nsc):
    blocks, bsz, v, s = att_weights.shape
    val3 = valid2all.reshape(bsz, 1, v)
    idx3 = idx_of_objs.reshape(bsz, 1, v)

    mask, signs = pl.pallas_call(
        _prep_body,
        grid=(nsc,),
        in_specs=[
            pl.BlockSpec((1, 1, v), lambda i: (i, 0, 0)),
            pl.BlockSpec((1, 1, v), lambda i: (i, 0, 0)),
            pl.BlockSpec((1, v, s), lambda i: (i, 0, 0)),
        ],
        out_specs=[
            pl.BlockSpec((1, 1, v), lambda i: (i, 0, 0)),
            pl.BlockSpec((1, v, s), lambda i: (i, 0, 0)),
        ],
        out_shape=[
            jax.ShapeDtypeStruct((nsc, 1, v), jnp.float32),
            jax.ShapeDtypeStruct((nsc, v, s), jnp.float32),
        ],
    )(val3, idx3, syb_graph)
    return mask.reshape(-1), signs.reshape(-1)


def _sc_partial(mask_flat, signs_flat, att_weights, nsc):
    blocks, bsz, v, s = att_weights.shape
    nw = SC_CORES * SC_SUBCORES
    cl = (nsc * v) // nw
    assert cl == LANES and v % cl == 0
    att_flat = att_weights.reshape(-1)

    mesh = plsc.VectorSubcoreMesh(core_axis_name="c", subcore_axis_name="s",
                                  num_cores=SC_CORES,
                                  num_subcores=SC_SUBCORES)

    @functools.partial(
        pl.kernel,
        out_type=jax.ShapeDtypeStruct((nw, LANES), jnp.float32),
        mesh=mesh,
        scratch_types=[
            pltpu.VMEM((2 * LANES,), jnp.float32),
            pltpu.VMEM((2 * LANES,), jnp.float32),
            pltpu.VMEM((cl * s,), jnp.float32),
            pltpu.VMEM((cl * s,), jnp.float32),
            pltpu.VMEM((cl * s,), jnp.float32),
            pltpu.VMEM((LANES,), jnp.float32),
            pltpu.SemaphoreType.DMA,
            pltpu.SemaphoreType.DMA,
            pltpu.SemaphoreType.DMA,
        ],
    )
    def sc_kernel(mask_hbm, sign_hbm, att_hbm, out_hbm,
                  msk_vm, red_vm, sgn_vm, att_a, att_b,
                  res_vm, sem_g, sem_a, sem_b):
        wid = lax.axis_index("s") * SC_CORES + lax.axis_index("c")
        base = wid * cl

        zv = jnp.zeros((LANES,), jnp.float32)
        pltpu.sync_copy(mask_hbm.at[pl.ds(base, cl)],
                        msk_vm.at[pl.ds(0, LANES)])
        msk_vm[pl.ds(LANES, LANES)] = zv
        red_vm[pl.ds(LANES, LANES)] = zv

        h_s = pltpu.async_copy(sign_hbm.at[pl.ds(base * s, cl * s)],
                               sgn_vm, sem_g)
        att_bufs = (att_a, att_b)
        att_sems = (sem_a, sem_b)
        handles = [None, None]
        handles[0] = pltpu.async_copy(
            att_hbm.at[pl.ds(base * s, cl * s)], att_bufs[0], att_sems[0])
        h_s.wait()

        lane = lax.iota(jnp.int32, LANES)
        e0f = (1 - jnp.minimum(lane, 1)).astype(jnp.float32)

        def k_body(att_ref, add_unfilled, k, lacc):
            def dot_body(c, acc):
                for u in range(8):
                    off = (c * 8 + u) * LANES
                    a = att_ref[pl.ds(k * s + off, LANES)]
                    sg = sgn_vm[pl.ds(k * s + off, LANES)]
                    acc = acc + a * sg
                return acc

            acc = lax.fori_loop(0, s // (8 * LANES), dot_body,
                                jnp.zeros((LANES,), jnp.float32))
            for sh in (8, 4, 2, 1):
                red_vm[pl.ds(0, LANES)] = acc
                acc = acc + red_vm[pl.ds(sh, LANES)]
            mk = msk_vm[pl.ds(k, LANES)]
            contrib = jnp.maximum(acc + MARGIN, 0.0) * e0f * mk
            if add_unfilled:
                contrib = contrib + (blocks * MARGIN) * e0f * (1.0 - mk)
            return lacc + contrib

        lacc = jnp.zeros((LANES,), jnp.float32)
        for blk in range(blocks):
            handles[blk % 2].wait()
            if blk + 1 < blocks:
                handles[(blk + 1) % 2] = pltpu.async_copy(
                    att_hbm.at[pl.ds(((blk + 1) * bsz * v + base) * s,
                                     cl * s)],
                    att_bufs[(blk + 1) % 2], att_sems[(blk + 1) % 2])
            lacc = lax.fori_loop(
                0, cl,
                functools.partial(k_body, att_bufs[blk % 2], blk == 0),
                lacc)

        res_vm[...] = lacc
        pltpu.sync_copy(res_vm, out_hbm.at[wid])

    out = sc_kernel(mask_flat, signs_flat, att_flat)
    return jnp.sum(out)


def kernel(idx_of_objs, valid2all, syb_graph, att_weights, vis_len):
    del vis_len
    blocks, bsz, v, s = att_weights.shape
    nsc = NSC

    partial = _tc_partial(idx_of_objs, valid2all, syb_graph, att_weights, nsc)
    if nsc > 0:
        mask_flat, signs_flat = _tc_prep(idx_of_objs, valid2all, syb_graph,
                                         att_weights, nsc)
        partial = partial + _sc_partial(mask_flat, signs_flat,
                                        att_weights, nsc)
    total = jnp.float32(blocks * bsz * v)
    return partial / total


# speedup vs baseline: 1.0017x; 1.0017x over previous
import functools

import jax
import jax.numpy as jnp
from jax import lax
from jax.experimental import pallas as pl
from jax.experimental.pallas import tpu as pltpu
from jax.experimental.pallas import tpu_sc as plsc

MARGIN = 0.6
NSC = 4
LANES = 16
SC_CORES = 2
SC_SUBCORES = 16


def _tc_body(val_ref, idxo_ref, syb_ref, att_ref, out_ref):
    blocks = att_ref.shape[0]
    v = syb_ref.shape[1]

    vrow = val_ref[0]
    iorow = idxo_ref[0]

    citer = lax.broadcasted_iota(jnp.int32, (v, v), 0)
    ovt = (jnp.broadcast_to(vrow, (v, v)) == citer).astype(jnp.float32)
    oit = (jnp.broadcast_to(iorow, (v, v)) == citer).astype(jnp.float32)

    eq = lax.dot_general(ovt, ovt, (((0,), (0,)), ((), ())),
                         preferred_element_type=jnp.float32)
    lprime = lax.broadcasted_iota(jnp.int32, (v, v), 1)
    first = jnp.min(jnp.where(eq > 0.5, lprime, v), axis=1, keepdims=True)
    lidx = lax.broadcasted_iota(jnp.int32, (v, 1), 0)
    m = (first == lidx).astype(jnp.float32)

    sybsign = 1.0 - 2.0 * syb_ref[0].astype(jnp.float32)
    gs = lax.dot_general(oit, sybsign, (((0,), (0,)), ((), ())),
                         preferred_element_type=jnp.float32)

    partial = jnp.float32(0.0)
    for b in range(blocks):
        d = jnp.sum(att_ref[b, 0] * gs, axis=1, keepdims=True)
        partial += jnp.sum(jnp.maximum(d + MARGIN, 0.0) * m)
    nfirst = jnp.sum(m)
    partial += blocks * (v - nfirst) * MARGIN

    tile = jnp.full((8, 128), partial, dtype=jnp.float32)
    i = pl.program_id(0)

    @pl.when(i == 0)
    def _init():
        out_ref[...] = tile

    @pl.when(i > 0)
    def _acc():
        out_ref[...] += tile


def _tc_partial(idx_of_objs, valid2all, syb_graph, att_weights, nsc):
    blocks, bsz, v, s = att_weights.shape
    nb = bsz - nsc
    val3 = valid2all.reshape(bsz, 1, v)
    idx3 = idx_of_objs.reshape(bsz, 1, v)

    out = pl.pallas_call(
        _tc_body,
        grid=(nb,),
        in_specs=[
            pl.BlockSpec((1, 1, v), lambda i: (i + nsc, 0, 0)),
            pl.BlockSpec((1, 1, v), lambda i: (i + nsc, 0, 0)),
            pl.BlockSpec((1, v, s), lambda i: (i + nsc, 0, 0)),
            pl.BlockSpec((blocks, 1, v, s), lambda i: (0, i + nsc, 0, 0)),
        ],
        out_specs=pl.BlockSpec((8, 128), lambda i: (0, 0)),
        out_shape=jax.ShapeDtypeStruct((8, 128), jnp.float32),
    )(val3, idx3, syb_graph, att_weights)
    return out[0, 0]


def _prep_body(val_ref, idxo_ref, syb_ref, m_ref, gs_ref):
    v = syb_ref.shape[1]
    vrow = val_ref[0]
    iorow = idxo_ref[0]

    citer = lax.broadcasted_iota(jnp.int32, (v, v), 0)
    ovt = (jnp.broadcast_to(vrow, (v, v)) == citer).astype(jnp.float32)
    oit = (jnp.broadcast_to(iorow, (v, v)) == citer).astype(jnp.float32)

    eq = lax.dot_general(ovt, ovt, (((0,), (0,)), ((), ())),
                         preferred_element_type=jnp.float32)
    lp0 = lax.broadcasted_iota(jnp.int32, (v, v), 0)
    first = jnp.min(jnp.where(eq > 0.5, lp0, v), axis=0, keepdims=True)
    laneio = lax.broadcasted_iota(jnp.int32, (1, v), 1)
    m_ref[0] = (first == laneio).astype(jnp.float32)

    sybsign = 1.0 - 2.0 * syb_ref[0].astype(jnp.float32)
    gs_ref[0] = lax.dot_general(oit, sybsign, (((0,), (0,)), ((), ())),
                                preferred_element_type=jnp.float32)


def _tc_prep(idx_of_objs, valid2all, syb_graph, att_weights, nsc):
    blocks, bsz, v, s = att_weights.shape
    val3 = valid2all.reshape(bsz, 1, v)
    idx3 = idx_of_objs.reshape(bsz, 1, v)

    mask, signs = pl.pallas_call(
        _prep_body,
        grid=(nsc,),
        in_specs=[
            pl.BlockSpec((1, 1, v), lambda i: (i, 0, 0)),
            pl.BlockSpec((1, 1, v), lambda i: (i, 0, 0)),
            pl.BlockSpec((1, v, s), lambda i: (i, 0, 0)),
        ],
        out_specs=[
            pl.BlockSpec((1, 1, v), lambda i: (i, 0, 0)),
            pl.BlockSpec((1, v, s), lambda i: (i, 0, 0)),
        ],
        out_shape=[
            jax.ShapeDtypeStruct((nsc, 1, v), jnp.float32),
            jax.ShapeDtypeStruct((nsc, v, s), jnp.float32),
        ],
    )(val3, idx3, syb_graph)
    return mask.reshape(-1), signs.reshape(-1)


def _sc_partial(mask_flat, signs_flat, att_weights, nsc):
    blocks, bsz, v, s = att_weights.shape
    nw = SC_CORES * SC_SUBCORES
    cl = (nsc * v) // nw
    assert cl == LANES and v % cl == 0
    att_flat = att_weights.reshape(-1)

    mesh = plsc.VectorSubcoreMesh(core_axis_name="c", subcore_axis_name="s",
                                  num_cores=SC_CORES,
                                  num_subcores=SC_SUBCORES)

    @functools.partial(
        pl.kernel,
        out_type=jax.ShapeDtypeStruct((nw, LANES), jnp.float32),
        mesh=mesh,
        scratch_types=[
            pltpu.VMEM((2 * LANES,), jnp.float32),
            pltpu.VMEM((2 * LANES,), jnp.float32),
            pltpu.VMEM((cl * s,), jnp.float32),
            pltpu.VMEM((cl * s,), jnp.float32),
            pltpu.VMEM((cl * s,), jnp.float32),
            pltpu.VMEM((LANES,), jnp.float32),
            pltpu.SemaphoreType.DMA,
            pltpu.SemaphoreType.DMA,
            pltpu.SemaphoreType.DMA,
        ],
    )
    def sc_kernel(mask_hbm, sign_hbm, att_hbm, out_hbm,
                  msk_vm, red_vm, sgn_vm, att_a, att_b,
                  res_vm, sem_g, sem_a, sem_b):
        wid = lax.axis_index("s") * SC_CORES + lax.axis_index("c")
        base = wid * cl

        zv = jnp.zeros((LANES,), jnp.float32)
        pltpu.sync_copy(mask_hbm.at[pl.ds(base, cl)],
                        msk_vm.at[pl.ds(0, LANES)])
        msk_vm[pl.ds(LANES, LANES)] = zv
        red_vm[pl.ds(LANES, LANES)] = zv

        h_s = pltpu.async_copy(sign_hbm.at[pl.ds(base * s, cl * s)],
                               sgn_vm, sem_g)
        att_bufs = (att_a, att_b)
        att_sems = (sem_a, sem_b)
        handles = [None, None]
        handles[0] = pltpu.async_copy(
            att_hbm.at[pl.ds(base * s, cl * s)], att_bufs[0], att_sems[0])
        h_s.wait()

        lane = lax.iota(jnp.int32, LANES)
        e0f = (1 - jnp.minimum(lane, 1)).astype(jnp.float32)

        def k_body(att_ref, add_unfilled, k, lacc):
            def dot_body(c, accs):
                accs = list(accs)
                for u in range(8):
                    off = (c * 8 + u) * LANES
                    a = att_ref[pl.ds(k * s + off, LANES)]
                    sg = sgn_vm[pl.ds(k * s + off, LANES)]
                    accs[u % 4] = accs[u % 4] + a * sg
                return tuple(accs)

            zf = jnp.zeros((LANES,), jnp.float32)
            a0, a1, a2, a3 = lax.fori_loop(0, s // (8 * LANES), dot_body,
                                           (zf, zf, zf, zf))
            acc = (a0 + a1) + (a2 + a3)
            for sh in (8, 4, 2, 1):
                red_vm[pl.ds(0, LANES)] = acc
                acc = acc + red_vm[pl.ds(sh, LANES)]
            mk = msk_vm[pl.ds(k, LANES)]
            contrib = jnp.maximum(acc + MARGIN, 0.0) * e0f * mk
            if add_unfilled:
                contrib = contrib + (blocks * MARGIN) * e0f * (1.0 - mk)
            return lacc + contrib

        lacc = jnp.zeros((LANES,), jnp.float32)
        for blk in range(blocks):
            handles[blk % 2].wait()
            if blk + 1 < blocks:
                handles[(blk + 1) % 2] = pltpu.async_copy(
                    att_hbm.at[pl.ds(((blk + 1) * bsz * v + base) * s,
                                     cl * s)],
                    att_bufs[(blk + 1) % 2], att_sems[(blk + 1) % 2])
            lacc = lax.fori_loop(
                0, cl,
                functools.partial(k_body, att_bufs[blk % 2], blk == 0),
                lacc)

        res_vm[...] = lacc
        pltpu.sync_copy(res_vm, out_hbm.at[wid])

    out = sc_kernel(mask_flat, signs_flat, att_flat)
    return jnp.sum(out)


def kernel(idx_of_objs, valid2all, syb_graph, att_weights, vis_len):
    del vis_len
    blocks, bsz, v, s = att_weights.shape
    nsc = NSC

    partial = _tc_partial(idx_of_objs, valid2all, syb_graph, att_weights, nsc)
    if nsc > 0:
        mask_flat, signs_flat = _tc_prep(idx_of_objs, valid2all, syb_graph,
                                         att_weights, nsc)
        partial = partial + _sc_partial(mask_flat, signs_flat,
                                        att_weights, nsc)
    total = jnp.float32(blocks * bsz * v)
    return partial / total


# speedup vs baseline: 1.0151x; 1.0133x over previous
import functools

import jax
import jax.numpy as jnp
from jax import lax
from jax.experimental import pallas as pl
from jax.experimental.pallas import tpu as pltpu
from jax.experimental.pallas import tpu_sc as plsc

MARGIN = 0.6
NSC = 2
LANES = 16
SC_CORES = 2
SC_SUBCORES = 16


def _tc_body(val_ref, idxo_ref, syb_ref, att_ref, out_ref):
    blocks = att_ref.shape[0]
    v = syb_ref.shape[1]

    vrow = val_ref[0]
    iorow = idxo_ref[0]

    citer = lax.broadcasted_iota(jnp.int32, (v, v), 0)
    ovt = (jnp.broadcast_to(vrow, (v, v)) == citer).astype(jnp.float32)
    oit = (jnp.broadcast_to(iorow, (v, v)) == citer).astype(jnp.float32)

    eq = lax.dot_general(ovt, ovt, (((0,), (0,)), ((), ())),
                         preferred_element_type=jnp.float32)
    lprime = lax.broadcasted_iota(jnp.int32, (v, v), 1)
    first = jnp.min(jnp.where(eq > 0.5, lprime, v), axis=1, keepdims=True)
    lidx = lax.broadcasted_iota(jnp.int32, (v, 1), 0)
    m = (first == lidx).astype(jnp.float32)

    sybsign = 1.0 - 2.0 * syb_ref[0].astype(jnp.float32)
    gs = lax.dot_general(oit, sybsign, (((0,), (0,)), ((), ())),
                         preferred_element_type=jnp.float32)

    partial = jnp.float32(0.0)
    for b in range(blocks):
        d = jnp.sum(att_ref[b, 0] * gs, axis=1, keepdims=True)
        partial += jnp.sum(jnp.maximum(d + MARGIN, 0.0) * m)
    nfirst = jnp.sum(m)
    partial += blocks * (v - nfirst) * MARGIN

    tile = jnp.full((8, 128), partial, dtype=jnp.float32)
    i = pl.program_id(0)

    @pl.when(i == 0)
    def _init():
        out_ref[...] = tile

    @pl.when(i > 0)
    def _acc():
        out_ref[...] += tile


def _tc_partial(idx_of_objs, valid2all, syb_graph, att_weights, nsc):
    blocks, bsz, v, s = att_weights.shape
    nb = bsz - nsc
    val3 = valid2all.reshape(bsz, 1, v)
    idx3 = idx_of_objs.reshape(bsz, 1, v)

    out = pl.pallas_call(
        _tc_body,
        grid=(nb,),
        in_specs=[
            pl.BlockSpec((1, 1, v), lambda i: (i + nsc, 0, 0)),
            pl.BlockSpec((1, 1, v), lambda i: (i + nsc, 0, 0)),
            pl.BlockSpec((1, v, s), lambda i: (i + nsc, 0, 0)),
            pl.BlockSpec((blocks, 1, v, s), lambda i: (0, i + nsc, 0, 0)),
        ],
        out_specs=pl.BlockSpec((8, 128), lambda i: (0, 0)),
        out_shape=jax.ShapeDtypeStruct((8, 128), jnp.float32),
    )(val3, idx3, syb_graph, att_weights)
    return out[0, 0]


def _prep_body(val_ref, idxo_ref, syb_ref, m_ref, gs_ref):
    v = syb_ref.shape[1]
    vrow = val_ref[0]
    iorow = idxo_ref[0]

    citer = lax.broadcasted_iota(jnp.int32, (v, v), 0)
    ovt = (jnp.broadcast_to(vrow, (v, v)) == citer).astype(jnp.float32)
    oit = (jnp.broadcast_to(iorow, (v, v)) == citer).astype(jnp.float32)

    eq = lax.dot_general(ovt, ovt, (((0,), (0,)), ((), ())),
                         preferred_element_type=jnp.float32)
    lp0 = lax.broadcasted_iota(jnp.int32, (v, v), 0)
    first = jnp.min(jnp.where(eq > 0.5, lp0, v), axis=0, keepdims=True)
    laneio = lax.broadcasted_iota(jnp.int32, (1, v), 1)
    m_ref[0] = (first == laneio).astype(jnp.float32)

    sybsign = 1.0 - 2.0 * syb_ref[0].astype(jnp.float32)
    gs_ref[0] = lax.dot_general(oit, sybsign, (((0,), (0,)), ((), ())),
                                preferred_element_type=jnp.float32)


def _tc_prep(idx_of_objs, valid2all, syb_graph, att_weights, nsc):
    blocks, bsz, v, s = att_weights.shape
    val3 = valid2all.reshape(bsz, 1, v)
    idx3 = idx_of_objs.reshape(bsz, 1, v)

    mask, signs = pl.pallas_call(
        _prep_body,
        grid=(nsc,),
        in_specs=[
            pl.BlockSpec((1, 1, v), lambda i: (i, 0, 0)),
            pl.BlockSpec((1, 1, v), lambda i: (i, 0, 0)),
            pl.BlockSpec((1, v, s), lambda i: (i, 0, 0)),
        ],
        out_specs=[
            pl.BlockSpec((1, 1, v), lambda i: (i, 0, 0)),
            pl.BlockSpec((1, v, s), lambda i: (i, 0, 0)),
        ],
        out_shape=[
            jax.ShapeDtypeStruct((nsc, 1, v), jnp.float32),
            jax.ShapeDtypeStruct((nsc, v, s), jnp.float32),
        ],
    )(val3, idx3, syb_graph)
    return mask.reshape(-1), signs.reshape(-1)


def _sc_partial(mask_flat, signs_flat, att_weights, nsc):
    blocks, bsz, v, s = att_weights.shape
    nw = SC_CORES * SC_SUBCORES
    cl = (nsc * v) // nw
    assert cl <= LANES and cl % 8 == 0 and v % cl == 0
    att_flat = att_weights.reshape(-1)

    mesh = plsc.VectorSubcoreMesh(core_axis_name="c", subcore_axis_name="s",
                                  num_cores=SC_CORES,
                                  num_subcores=SC_SUBCORES)

    @functools.partial(
        pl.kernel,
        out_type=jax.ShapeDtypeStruct((nw, LANES), jnp.float32),
        mesh=mesh,
        scratch_types=[
            pltpu.VMEM((2 * LANES,), jnp.float32),
            pltpu.VMEM((2 * LANES,), jnp.float32),
            pltpu.VMEM((cl * s,), jnp.float32),
            pltpu.VMEM((cl * s,), jnp.float32),
            pltpu.VMEM((cl * s,), jnp.float32),
            pltpu.VMEM((LANES,), jnp.float32),
            pltpu.SemaphoreType.DMA,
            pltpu.SemaphoreType.DMA,
            pltpu.SemaphoreType.DMA,
        ],
    )
    def sc_kernel(mask_hbm, sign_hbm, att_hbm, out_hbm,
                  msk_vm, red_vm, sgn_vm, att_a, att_b,
                  res_vm, sem_g, sem_a, sem_b):
        wid = lax.axis_index("s") * SC_CORES + lax.axis_index("c")
        base = wid * cl

        zv = jnp.zeros((LANES,), jnp.float32)
        msk_vm[pl.ds(0, LANES)] = zv
        msk_vm[pl.ds(LANES, LANES)] = zv
        red_vm[pl.ds(LANES, LANES)] = zv
        pltpu.sync_copy(mask_hbm.at[pl.ds(base, cl)],
                        msk_vm.at[pl.ds(0, cl)])

        h_s = pltpu.async_copy(sign_hbm.at[pl.ds(base * s, cl * s)],
                               sgn_vm, sem_g)
        att_bufs = (att_a, att_b)
        att_sems = (sem_a, sem_b)
        handles = [None, None]
        handles[0] = pltpu.async_copy(
            att_hbm.at[pl.ds(base * s, cl * s)], att_bufs[0], att_sems[0])
        h_s.wait()

        lane = lax.iota(jnp.int32, LANES)
        e0f = (1 - jnp.minimum(lane, 1)).astype(jnp.float32)

        def k_body(att_ref, add_unfilled, k, lacc):
            def dot_body(c, accs):
                accs = list(accs)
                for u in range(8):
                    off = (c * 8 + u) * LANES
                    a = att_ref[pl.ds(k * s + off, LANES)]
                    sg = sgn_vm[pl.ds(k * s + off, LANES)]
                    accs[u % 4] = accs[u % 4] + a * sg
                return tuple(accs)

            zf = jnp.zeros((LANES,), jnp.float32)
            a0, a1, a2, a3 = lax.fori_loop(0, s // (8 * LANES), dot_body,
                                           (zf, zf, zf, zf))
            acc = (a0 + a1) + (a2 + a3)
            for sh in (8, 4, 2, 1):
                red_vm[pl.ds(0, LANES)] = acc
                acc = acc + red_vm[pl.ds(sh, LANES)]
            mk = msk_vm[pl.ds(k, LANES)]
            contrib = jnp.maximum(acc + MARGIN, 0.0) * e0f * mk
            if add_unfilled:
                contrib = contrib + (blocks * MARGIN) * e0f * (1.0 - mk)
            return lacc + contrib

        lacc = jnp.zeros((LANES,), jnp.float32)
        for blk in range(blocks):
            handles[blk % 2].wait()
            if blk + 1 < blocks:
                handles[(blk + 1) % 2] = pltpu.async_copy(
                    att_hbm.at[pl.ds(((blk + 1) * bsz * v + base) * s,
                                     cl * s)],
                    att_bufs[(blk + 1) % 2], att_sems[(blk + 1) % 2])
            lacc = lax.fori_loop(
                0, cl,
                functools.partial(k_body, att_bufs[blk % 2], blk == 0),
                lacc)

        res_vm[...] = lacc
        pltpu.sync_copy(res_vm, out_hbm.at[wid])

    out = sc_kernel(mask_flat, signs_flat, att_flat)
    return jnp.sum(out)


def kernel(idx_of_objs, valid2all, syb_graph, att_weights, vis_len):
    del vis_len
    blocks, bsz, v, s = att_weights.shape
    nsc = NSC

    partial = _tc_partial(idx_of_objs, valid2all, syb_graph, att_weights, nsc)
    if nsc > 0:
        mask_flat, signs_flat = _tc_prep(idx_of_objs, valid2all, syb_graph,
                                         att_weights, nsc)
        partial = partial + _sc_partial(mask_flat, signs_flat,
                                        att_weights, nsc)
    total = jnp.float32(blocks * bsz * v)
    return partial / total
